# Initial kernel scaffold; baseline (speedup 1.0000x reference)
import jax
import jax.numpy as jnp
from jax.experimental import pallas as pl


def kernel(x, emb_table, pos_table):
    raise NotImplementedError("write your pallas kernel here")


# speedup vs baseline: 2.0474x; 2.0474x over previous
import functools

import jax
import jax.numpy as jnp
from jax import lax
from jax.experimental import pallas as pl
from jax.experimental.pallas import tpu as pltpu
from jax.experimental.pallas import tpu_sc as plsc

VOCAB = 100000
EMB = 64
B = 4096
L = 200

N_TOK = B * L
CHUNK = 128
N_CHUNKS = N_TOK // CHUNK
NW = 32
CPW = N_CHUNKS // NW


def _body(x_hbm, emb_hbm, pos_hbm, out_hbm, idx_v, rows_v, pos2_v, sem):
    wid = lax.axis_index("s") * 2 + lax.axis_index("c")
    base = wid * CPW

    pltpu.sync_copy(pos_hbm.at[pl.ds(0, L)], pos2_v.at[pl.ds(0, L)])
    pltpu.sync_copy(pos_hbm.at[pl.ds(0, L)], pos2_v.at[pl.ds(L, L)])

    @pl.loop(0, CPW)
    def _chunk(t):
        c = base + t
        pltpu.sync_copy(x_hbm.at[pl.ds(c * CHUNK, CHUNK)], idx_v)
        pltpu.async_copy(emb_hbm.at[idx_v], rows_v, sem).wait()
        p0 = lax.rem(c * CHUNK, L)

        @pl.loop(0, CHUNK)
        def _row(r):
            pr = p0 + r
            for cc in range(EMB // 16):
                s = pl.ds(cc * 16, 16)
                rows_v[r, s] = rows_v[r, s] + pos2_v[pr, s]

        pltpu.sync_copy(rows_v, out_hbm.at[pl.ds(c * CHUNK, CHUNK)])


@jax.jit
def kernel(x, emb_table, pos_table):
    x_flat = jnp.reshape(x, (N_TOK,))
    mesh = plsc.VectorSubcoreMesh(core_axis_name="c", subcore_axis_name="s")
    out = pl.kernel(
        _body,
        out_type=jax.ShapeDtypeStruct((N_TOK, EMB), jnp.float32),
        mesh=mesh,
        compiler_params=pltpu.CompilerParams(use_tc_tiling_on_sc=False),
        scratch_types=[
            pltpu.VMEM((CHUNK,), jnp.int32),
            pltpu.VMEM((CHUNK, EMB), jnp.float32),
            pltpu.VMEM((2 * L, EMB), jnp.float32),
            pltpu.SemaphoreType.DMA,
        ],
    )(x_flat, emb_table, pos_table)
    return jnp.reshape(out, (B, L, EMB))


# speedup vs baseline: 2.9522x; 1.4420x over previous
import functools

import jax
import jax.numpy as jnp
from jax import lax
from jax.experimental import pallas as pl
from jax.experimental.pallas import tpu as pltpu
from jax.experimental.pallas import tpu_sc as plsc

VOCAB = 100000
EMB = 64
B = 4096
L = 200

N_TOK = B * L
CHUNK = 128
N_CHUNKS = N_TOK // CHUNK
NW = 32
CPW = N_CHUNKS // NW


def _body(x_hbm, emb_hbm, pos_hbm, out_hbm, idx_v, rows_v, pos_sh, sem):
    sid = lax.axis_index("s")
    wid = sid * 2 + lax.axis_index("c")
    base = wid * CPW

    @pl.when(sid == 0)
    def _init():
        pltpu.sync_copy(pos_hbm.at[pl.ds(0, L)], pos_sh.at[pl.ds(0, L)])
        pltpu.sync_copy(pos_hbm.at[pl.ds(0, L)], pos_sh.at[pl.ds(L, L)])

    plsc.subcore_barrier()

    @pl.loop(0, CPW)
    def _chunk(t):
        c = base + t
        p0 = lax.rem(c * CHUNK, L)
        pltpu.sync_copy(pos_sh.at[pl.ds(p0, CHUNK)], rows_v)
        pltpu.sync_copy(x_hbm.at[pl.ds(c * CHUNK, CHUNK)], idx_v)
        pltpu.async_copy(emb_hbm.at[idx_v], rows_v, sem, add=True).wait()
        pltpu.sync_copy(rows_v, out_hbm.at[pl.ds(c * CHUNK, CHUNK)])


@jax.jit
def kernel(x, emb_table, pos_table):
    x_flat = jnp.reshape(x, (N_TOK,))
    mesh = plsc.VectorSubcoreMesh(core_axis_name="c", subcore_axis_name="s")
    out = pl.kernel(
        _body,
        out_type=jax.ShapeDtypeStruct((N_TOK, EMB), jnp.float32),
        mesh=mesh,
        compiler_params=pltpu.CompilerParams(use_tc_tiling_on_sc=False),
        scratch_types=[
            pltpu.VMEM((CHUNK,), jnp.int32),
            pltpu.VMEM((CHUNK, EMB), jnp.float32),
            pltpu.VMEM_SHARED((2 * L, EMB), jnp.float32),
            pltpu.SemaphoreType.DMA,
        ],
    )(x_flat, emb_table, pos_table)
    return jnp.reshape(out, (B, L, EMB))


# speedup vs baseline: 3.5277x; 1.1949x over previous
import functools

import jax
import jax.numpy as jnp
from jax import lax
from jax.experimental import pallas as pl
from jax.experimental.pallas import tpu as pltpu
from jax.experimental.pallas import tpu_sc as plsc

VOCAB = 100000
EMB = 64
B = 4096
L = 200

N_TOK = B * L
CHUNK = 128
N_CHUNKS = N_TOK // CHUNK
NW = 32
CPW = N_CHUNKS // NW


def _body(x_hbm, emb_hbm, pos_hbm, out_hbm,
          idx0, idx1, rows0, rows1, pos_sh,
          sem_p0, sem_p1, sem_i0, sem_i1, sem_g):
    idx_v = (idx0, idx1)
    rows_v = (rows0, rows1)
    sem_p = (sem_p0, sem_p1)
    sem_i = (sem_i0, sem_i1)

    sid = lax.axis_index("s")
    wid = sid * 2 + lax.axis_index("c")
    base = wid * CPW
    end = base + CPW

    @pl.when(sid == 0)
    def _init():
        pltpu.sync_copy(pos_hbm.at[pl.ds(0, L)], pos_sh.at[pl.ds(0, L)])
        pltpu.sync_copy(pos_hbm.at[pl.ds(0, L)], pos_sh.at[pl.ds(L, L)])

    plsc.subcore_barrier()

    def start_load(c, b):
        p0 = lax.rem(c * CHUNK, L)
        pltpu.async_copy(pos_sh.at[pl.ds(p0, CHUNK)], rows_v[b], sem_p[b])
        pltpu.async_copy(x_hbm.at[pl.ds(c * CHUNK, CHUNK)], idx_v[b], sem_i[b])

    def wait_load(b):
        pltpu.make_async_copy(pos_sh.at[pl.ds(0, CHUNK)], rows_v[b], sem_p[b]).wait()
        pltpu.make_async_copy(x_hbm.at[pl.ds(0, CHUNK)], idx_v[b], sem_i[b]).wait()

    start_load(base, 0)

    @pl.loop(0, CPW, step=2)
    def _chunk(t):
        for db in range(2):
            c = base + t + db
            b = db
            o = 1 - db

            start_load(lax.min(c + 1, end - 1), o)

            wait_load(b)
            pltpu.async_copy(emb_hbm.at[idx_v[b]], rows_v[b], sem_g, add=True).wait()
            pltpu.sync_copy(rows_v[b], out_hbm.at[pl.ds(c * CHUNK, CHUNK)])

    wait_load(0)


@jax.jit
def kernel(x, emb_table, pos_table):
    x_flat = jnp.reshape(x, (N_TOK,))
    mesh = plsc.VectorSubcoreMesh(core_axis_name="c", subcore_axis_name="s")
    out = pl.kernel(
        _body,
        out_type=jax.ShapeDtypeStruct((N_TOK, EMB), jnp.float32),
        mesh=mesh,
        compiler_params=pltpu.CompilerParams(use_tc_tiling_on_sc=False),
        scratch_types=[
            pltpu.VMEM((CHUNK,), jnp.int32),
            pltpu.VMEM((CHUNK,), jnp.int32),
            pltpu.VMEM((CHUNK, EMB), jnp.float32),
            pltpu.VMEM((CHUNK, EMB), jnp.float32),
            pltpu.VMEM_SHARED((2 * L, EMB), jnp.float32),
            pltpu.SemaphoreType.DMA,
            pltpu.SemaphoreType.DMA,
            pltpu.SemaphoreType.DMA,
            pltpu.SemaphoreType.DMA,
            pltpu.SemaphoreType.DMA,
        ],
    )(x_flat, emb_table, pos_table)
    return jnp.reshape(out, (B, L, EMB))
